# baseline (speedup 1.0000x reference)
import jax
import jax.numpy as jnp
from jax.experimental import pallas as pl

N = 131072
TN = 2048


def _mlp_kernel(x_ref, w1_ref, b1_ref, w2_ref, b2_ref, w3_ref, b3_ref,
                w4_ref, b4_ref, out_ref):
    x = x_ref[...].astype(jnp.bfloat16)
    h = jnp.dot(x, w1_ref[...], preferred_element_type=jnp.float32)
    h = jnp.maximum(h + b1_ref[...], 0.0).astype(jnp.bfloat16)
    h = jnp.dot(h, w2_ref[...], preferred_element_type=jnp.float32)
    h = jnp.maximum(h + b2_ref[...], 0.0).astype(jnp.bfloat16)
    h = jnp.dot(h, w3_ref[...], preferred_element_type=jnp.float32)
    h = jnp.maximum(h + b3_ref[...], 0.0).astype(jnp.bfloat16)
    logits = jnp.dot(h, w4_ref[...], preferred_element_type=jnp.float32)
    logits = logits + b4_ref[...]
    m = jnp.max(logits, axis=-1, keepdims=True)
    e = jnp.exp(logits - m)
    out_ref[...] = e / jnp.sum(e, axis=-1, keepdims=True)


def kernel(select_feature, W1, b1, W2, b2, W3, b3, W4, b4):
    w1 = W1.astype(jnp.bfloat16)
    w2 = W2.astype(jnp.bfloat16)
    w3 = W3.astype(jnp.bfloat16)
    w4 = W4.astype(jnp.bfloat16)
    b1r = b1.reshape(1, -1)
    b2r = b2.reshape(1, -1)
    b3r = b3.reshape(1, -1)
    b4r = b4.reshape(1, -1)

    fixed = lambda i: (0, 0)
    grid = N // TN
    return pl.pallas_call(
        _mlp_kernel,
        grid=(grid,),
        in_specs=[
            pl.BlockSpec((TN, 512), lambda i: (i, 0)),
            pl.BlockSpec((512, 256), fixed),
            pl.BlockSpec((1, 256), fixed),
            pl.BlockSpec((256, 128), fixed),
            pl.BlockSpec((1, 128), fixed),
            pl.BlockSpec((128, 64), fixed),
            pl.BlockSpec((1, 64), fixed),
            pl.BlockSpec((64, 2), fixed),
            pl.BlockSpec((1, 2), fixed),
        ],
        out_specs=pl.BlockSpec((TN, 2), lambda i: (i, 0)),
        out_shape=jax.ShapeDtypeStruct((N, 2), jnp.float32),
    )(select_feature, w1, b1r, w2, b2r, w3, b3r, w4, b4r)


# speedup vs baseline: 1.0357x; 1.0357x over previous
import jax
import jax.numpy as jnp
from jax.experimental import pallas as pl
from jax.experimental.pallas import tpu as pltpu

N = 131072
TN = 2048


def _mlp_kernel(x_ref, w1_ref, b1_ref, w2_ref, b2_ref, w3_ref, b3_ref,
                w4_ref, b4_ref, out_ref):
    x = x_ref[...].astype(jnp.bfloat16)
    h = jnp.dot(x, w1_ref[...], preferred_element_type=jnp.float32)
    h = jnp.maximum(h + b1_ref[...], 0.0).astype(jnp.bfloat16)
    h = jnp.dot(h, w2_ref[...], preferred_element_type=jnp.float32)
    h = jnp.maximum(h + b2_ref[...], 0.0).astype(jnp.bfloat16)
    h = jnp.dot(h, w3_ref[...], preferred_element_type=jnp.float32)
    h = jnp.maximum(h + b3_ref[...], 0.0).astype(jnp.bfloat16)
    d = jnp.dot(h, w4_ref[...], preferred_element_type=jnp.float32)
    d = d + b4_ref[...]
    out_ref[...] = 1.0 / (1.0 + jnp.exp(-d))


def kernel(select_feature, W1, b1, W2, b2, W3, b3, W4, b4):
    w1 = W1.astype(jnp.bfloat16)
    w2 = W2.astype(jnp.bfloat16)
    w3 = W3.astype(jnp.bfloat16)
    w4d = jnp.stack([W4[:, 0] - W4[:, 1], W4[:, 1] - W4[:, 0]], axis=1)
    b4d = jnp.stack([b4[0] - b4[1], b4[1] - b4[0]]).reshape(1, 2)
    w4 = w4d.astype(jnp.bfloat16)
    b1r = b1.reshape(1, -1)
    b2r = b2.reshape(1, -1)
    b3r = b3.reshape(1, -1)

    fixed = lambda i: (0, 0)
    grid = N // TN
    return pl.pallas_call(
        _mlp_kernel,
        grid=(grid,),
        in_specs=[
            pl.BlockSpec((TN, 512), lambda i: (i, 0)),
            pl.BlockSpec((512, 256), fixed),
            pl.BlockSpec((1, 256), fixed),
            pl.BlockSpec((256, 128), fixed),
            pl.BlockSpec((1, 128), fixed),
            pl.BlockSpec((128, 64), fixed),
            pl.BlockSpec((1, 64), fixed),
            pl.BlockSpec((64, 2), fixed),
            pl.BlockSpec((1, 2), fixed),
        ],
        out_specs=pl.BlockSpec((TN, 2), lambda i: (i, 0)),
        out_shape=jax.ShapeDtypeStruct((N, 2), jnp.float32),
        compiler_params=pltpu.CompilerParams(
            dimension_semantics=("parallel",),
        ),
    )(select_feature, w1, b1r, w2, b2r, w3, b3r, w4, b4d)
